# Initial kernel scaffold; baseline (speedup 1.0000x reference)
import jax
import jax.numpy as jnp
from jax.experimental import pallas as pl


def kernel(x, W, b, codebook):
    raise NotImplementedError("write your pallas kernel here")


# speedup vs baseline: 1.2148x; 1.2148x over previous
import functools

import jax
import jax.numpy as jnp
from jax import lax
from jax.experimental import pallas as pl
from jax.experimental.pallas import tpu as pltpu
from jax.experimental.pallas import tpu_sc as plsc

G = 2
V = 320
TB = 256


def _tc_body(x_ref, w_ref, b_ref, idx0_ref, idx1_ref, cppl_ref, pppl_ref,
             cnt_ref, psum_ref, *, n_tokens):
    pid = pl.program_id(0)
    nprog = pl.num_programs(0)

    @pl.when(pid == 0)
    def _init():
        cnt_ref[...] = jnp.zeros_like(cnt_ref)
        psum_ref[...] = jnp.zeros_like(psum_ref)

    logits = jnp.dot(x_ref[...], w_ref[...],
                     preferred_element_type=jnp.float32) + b_ref[...]

    for g, idx_ref in ((0, idx0_ref), (1, idx1_ref)):
        lg = logits[:, g * V:(g + 1) * V]
        m = jnp.max(lg, axis=-1, keepdims=True)
        iota = lax.broadcasted_iota(jnp.int32, lg.shape, 1)
        amax = jnp.min(jnp.where(lg == m, iota, V),
                       axis=-1, keepdims=True)
        idx_ref[...] = amax + g * V
        e = jnp.exp(lg - m)
        p = e / jnp.sum(e, axis=-1, keepdims=True)
        psum_ref[g:g + 1, :] += jnp.sum(p, axis=0, keepdims=True)
        onehot = (iota == amax).astype(jnp.float32)
        cnt_ref[g:g + 1, :] += jnp.sum(onehot, axis=0, keepdims=True)

    @pl.when(pid == nprog - 1)
    def _finish():
        hp = cnt_ref[0:G, :] / n_tokens
        ap = psum_ref[0:G, :] / n_tokens
        ent_h = jnp.sum(hp * jnp.log(hp + 1e-7), axis=-1, keepdims=True)
        ent_a = jnp.sum(ap * jnp.log(ap + 1e-7), axis=-1, keepdims=True)
        cppl_ref[...] = jnp.sum(jnp.exp(-ent_h), axis=0, keepdims=True)
        pppl_ref[...] = jnp.sum(jnp.exp(-ent_a), axis=0, keepdims=True)


def _tc_stage(flat, W, b2):
    n_tokens, C = flat.shape
    grid = n_tokens // TB
    body = functools.partial(_tc_body, n_tokens=float(n_tokens))
    return pl.pallas_call(
        body,
        grid=(grid,),
        in_specs=[
            pl.BlockSpec((TB, C), lambda i: (i, 0)),
            pl.BlockSpec((C, G * V), lambda i: (0, 0)),
            pl.BlockSpec((1, G * V), lambda i: (0, 0)),
        ],
        out_specs=[
            pl.BlockSpec((TB, 1), lambda i: (i, 0)),
            pl.BlockSpec((TB, 1), lambda i: (i, 0)),
            pl.BlockSpec((1, 1), lambda i: (0, 0)),
            pl.BlockSpec((1, 1), lambda i: (0, 0)),
        ],
        out_shape=[
            jax.ShapeDtypeStruct((n_tokens, 1), jnp.int32),
            jax.ShapeDtypeStruct((n_tokens, 1), jnp.int32),
            jax.ShapeDtypeStruct((1, 1), jnp.float32),
            jax.ShapeDtypeStruct((1, 1), jnp.float32),
        ],
        scratch_shapes=[
            pltpu.VMEM((8, V), jnp.float32),
            pltpu.VMEM((8, V), jnp.float32),
        ],
        compiler_params=pltpu.CompilerParams(
            dimension_semantics=("arbitrary",)),
    )(flat, W, b2)


def _sc_gather(table, idx):
    B = idx.shape[0]
    D = table.shape[1]
    info = plsc.get_sparse_core_info()
    nw = info.num_cores * info.num_subcores
    b_per_w = B // nw
    mesh = plsc.VectorSubcoreMesh(core_axis_name="c", subcore_axis_name="s")

    @functools.partial(
        pl.kernel, mesh=mesh,
        out_type=jax.ShapeDtypeStruct((B, D), jnp.float32),
        scratch_types=[
            pltpu.VMEM((b_per_w,), jnp.int32),
            pltpu.VMEM((b_per_w, D), jnp.float32),
            pltpu.SemaphoreType.DMA,
        ],
    )
    def k(table_hbm, idx_hbm, out_hbm, idx_v, rows_v, sem):
        wid = lax.axis_index("s") * info.num_cores + lax.axis_index("c")
        base = wid * b_per_w
        pltpu.sync_copy(idx_hbm.at[pl.ds(base, b_per_w)], idx_v)
        pltpu.async_copy(table_hbm.at[idx_v], rows_v, sem).wait()
        pltpu.sync_copy(rows_v, out_hbm.at[pl.ds(base, b_per_w)])

    return k(table, idx)


def kernel(x, W, b, codebook):
    bsz, tsz, fsz = x.shape
    flat = x.reshape(bsz * tsz, fsz)
    idx0, idx1, cppl, pppl = _tc_stage(flat, W, b.reshape(1, -1))
    idx = jnp.concatenate([idx0, idx1], axis=1).reshape(-1)
    table = codebook.reshape(G * V, -1)
    xq = _sc_gather(table, idx).reshape(bsz, tsz, -1)
    return xq, cppl.reshape(()), pppl.reshape(())
